# baseline (speedup 1.0000x reference)
import jax
import jax.numpy as jnp
from jax.experimental import pallas as pl
from jax.experimental.pallas import tpu as pltpu

_IOU_THRESHOLD = 0.8
_K_SELECT = 1024
_N = 20000
_NB = 20
_BR = 8
_ROWS = _NB * _BR
_COLS = 128
_BLK = _BR * _COLS
_N_PAD = _ROWS * _COLS
_NEG_INF = float("-inf")
_FAR = -1.0e30


def _precede(ka, ia, kb, ib):
    return (ka > kb) | ((ka == kb) & (ia < ib))


_GB = 4
_GR = _GB * _BR


def _sort_body(sc_ref, x1_ref, y1_ref, x2_ref, y2_ref,
               ks_ref, xs1_ref, ys1_ref, xs2_ref, ys2_ref):
    f = ((jax.lax.broadcasted_iota(jnp.int32, (_GR, _COLS), 0) % _BR) * _COLS
         + jax.lax.broadcasted_iota(jnp.int32, (_GR, _COLS), 1))
    arrs = [sc_ref[...], x1_ref[...], y1_ref[...], x2_ref[...], y2_ref[...],
            f]

    def partner_lane(a, j):
        return jnp.where((f & j) == 0,
                         jnp.roll(a, -j, axis=1), jnp.roll(a, j, axis=1))

    def partner_row(a, jr):
        g = a.reshape(_GR // (2 * jr), 2, jr, _COLS)
        sw = jnp.concatenate([g[:, 1:2], g[:, 0:1]], axis=1)
        return sw.reshape(_GR, _COLS)

    for k in [2, 4, 8, 16, 32, 64, 128, 256, 512, 1024]:
        j = k // 2
        while j >= 1:
            if j >= _COLS:
                jr = j // _COLS
                parts = [partner_row(a, jr) for a in arrs]
            else:
                parts = [partner_lane(a, j) for a in arrs]
            pk, pi = parts[0], parts[5]
            kk, ii = arrs[0], arrs[5]
            prec = _precede(kk, ii, pk, pi)
            is_upper = (f & j) != 0
            ascending = (f & k) == 0
            keep_mine = prec ^ is_upper ^ jnp.logical_not(ascending)
            arrs = [jnp.where(keep_mine, a, p) for a, p in zip(arrs, parts)]
            j //= 2

    ks_ref[...] = arrs[0]
    xs1_ref[...] = arrs[1]
    ys1_ref[...] = arrs[2]
    xs2_ref[...] = arrs[3]
    ys2_ref[...] = arrs[4]


def _block_sort(sc, x1, y1, x2, y2):
    spec = pl.BlockSpec((_GR, _COLS), lambda b: (b, 0))
    vreg = jax.ShapeDtypeStruct((_ROWS, _COLS), jnp.float32)
    return pl.pallas_call(
        _sort_body,
        grid=(_NB // _GB,),
        in_specs=[spec] * 5,
        out_specs=[spec] * 5,
        out_shape=[vreg] * 5,
    )(sc, x1, y1, x2, y2)


def _scan_body(ks_ref, xs1_ref, ys1_ref, xs2_ref, ys2_ref, b0_ref,
               ox1_ref, oy1_ref, ox2_ref, oy2_ref, osc_ref,
               h_ref, p_ref, hx1_ref, hy1_ref, hx2_ref, hy2_ref,
               kx1_ref, ky1_ref, kx2_ref, ky2_ref,
               ka_ref, ksc_ref):
    sub2 = jax.lax.broadcasted_iota(jnp.int32, (_BR, _COLS), 0)
    lane2 = jax.lax.broadcasted_iota(jnp.int32, (_BR, _COLS), 1)
    lane1 = jax.lax.broadcasted_iota(jnp.int32, (1, _COLS), 1)
    kidx = sub2 * _COLS + lane2

    b0x1 = jnp.max(b0_ref[0:1, :])
    b0y1 = jnp.max(b0_ref[1:2, :])
    b0x2 = jnp.max(b0_ref[2:3, :])
    b0y2 = jnp.max(b0_ref[3:4, :])
    b0sc = jnp.max(b0_ref[4:5, :])
    b0area = (b0x2 - b0x1) * (b0y2 - b0y1)

    far = jnp.full((_BR, _COLS), _FAR, jnp.float32)
    zero = jnp.zeros((_BR, _COLS), jnp.float32)
    kx1_ref[...] = far
    ky1_ref[...] = far
    kx2_ref[...] = far
    ky2_ref[...] = far
    ka_ref[...] = zero
    ksc_ref[...] = zero
    p_ref[...] = jnp.zeros((1, _COLS), jnp.int32)
    h_ref[...] = jnp.full((1, _COLS), _NEG_INF, jnp.float32)
    hx1_ref[...] = jnp.zeros((1, _COLS), jnp.float32)
    hy1_ref[...] = jnp.zeros((1, _COLS), jnp.float32)
    hx2_ref[...] = jnp.zeros((1, _COLS), jnp.float32)
    hy2_ref[...] = jnp.zeros((1, _COLS), jnp.float32)

    def lext(ref, row, ln):
        return jnp.max(jnp.where(lane1 == ln, ref[pl.ds(row, 1), :], _NEG_INF))

    def inith(b, carry):
        sel = lane1 == b
        h_ref[...] = jnp.where(sel, lext(ks_ref, _BR * b, 0), h_ref[...])
        hx1_ref[...] = jnp.where(sel, lext(xs1_ref, _BR * b, 0), hx1_ref[...])
        hy1_ref[...] = jnp.where(sel, lext(ys1_ref, _BR * b, 0), hy1_ref[...])
        hx2_ref[...] = jnp.where(sel, lext(xs2_ref, _BR * b, 0), hx2_ref[...])
        hy2_ref[...] = jnp.where(sel, lext(ys2_ref, _BR * b, 0), hy2_ref[...])
        return carry

    jax.lax.fori_loop(0, _NB, inith, 0)

    def produce():
        h = h_ref[...]
        m = jnp.max(h)
        b = jnp.min(jnp.where(h == m, lane1, _COLS))
        sel = lane1 == b
        ninf = jnp.float32(_NEG_INF)
        cx1 = jnp.max(jnp.where(sel, hx1_ref[...], ninf))
        cy1 = jnp.max(jnp.where(sel, hy1_ref[...], ninf))
        cx2 = jnp.max(jnp.where(sel, hx2_ref[...], ninf))
        cy2 = jnp.max(jnp.where(sel, hy2_ref[...], ninf))
        pv = p_ref[...]
        pn = jnp.max(jnp.where(sel, pv, -1)) + 1
        p_ref[...] = jnp.where(sel, pn, pv)
        pnc = jnp.minimum(pn, _BLK - 1)
        rown = _BR * b + pnc // _COLS
        lanen = pnc % _COLS
        dead = pn > _BLK - 1
        nh = jnp.where(dead, ninf, lext(ks_ref, rown, lanen))
        h_ref[...] = jnp.where(sel, nh, h)
        hx1_ref[...] = jnp.where(sel, lext(xs1_ref, rown, lanen), hx1_ref[...])
        hy1_ref[...] = jnp.where(sel, lext(ys1_ref, rown, lanen), hy1_ref[...])
        hx2_ref[...] = jnp.where(sel, lext(xs2_ref, rown, lanen), hx2_ref[...])
        hy2_ref[...] = jnp.where(sel, lext(ys2_ref, rown, lanen), hy2_ref[...])
        return m, cx1, cy1, cx2, cy2

    def body(state):
        count, key, cx1, cy1, cx2, cy2 = state
        area_c = (cx2 - cx1) * (cy2 - cy1)
        kx1 = kx1_ref[...]
        ky1 = ky1_ref[...]
        kx2 = kx2_ref[...]
        ky2 = ky2_ref[...]
        ka = ka_ref[...]
        xx1 = jnp.maximum(kx1, cx1)
        yy1 = jnp.maximum(ky1, cy1)
        xx2 = jnp.minimum(kx2, cx2)
        yy2 = jnp.minimum(ky2, cy2)
        inter = jnp.maximum(xx2 - xx1, 0.0) * jnp.maximum(yy2 - yy1, 0.0)
        iou = inter / (ka + area_c - inter + 1e-8)
        suppressed = jnp.max(jnp.where(iou > _IOU_THRESHOLD, 1.0, 0.0)) > 0.0
        exh = key == _NEG_INF
        accept = jnp.logical_or(jnp.logical_not(suppressed), exh)
        fx1 = jnp.where(exh, b0x1, cx1)
        fy1 = jnp.where(exh, b0y1, cy1)
        fx2 = jnp.where(exh, b0x2, cx2)
        fy2 = jnp.where(exh, b0y2, cy2)
        fsc = jnp.where(exh, b0sc, key)
        fa = jnp.where(exh, b0area, area_c)
        write = jnp.logical_and(accept, kidx == count)
        kx1_ref[...] = jnp.where(write, fx1, kx1)
        ky1_ref[...] = jnp.where(write, fy1, ky1)
        kx2_ref[...] = jnp.where(write, fx2, kx2)
        ky2_ref[...] = jnp.where(write, fy2, ky2)
        ka_ref[...] = jnp.where(write, fa, ka)
        ksc_ref[...] = jnp.where(write, fsc, ksc_ref[...])
        ncount = count + jnp.where(accept, 1, 0)
        nkey, nx1, ny1, nx2, ny2 = produce()
        return (ncount, nkey, nx1, ny1, nx2, ny2)

    def body4(state):
        return body(body(body(body(state))))

    first = produce()
    state0 = (jnp.int32(0),) + first
    jax.lax.while_loop(lambda s: s[0] < _K_SELECT, body4, state0)

    ox1_ref[...] = kx1_ref[...]
    oy1_ref[...] = ky1_ref[...]
    ox2_ref[...] = kx2_ref[...]
    oy2_ref[...] = ky2_ref[...]
    osc_ref[...] = ksc_ref[...]


def kernel(boxes, scores):
    pad = _N_PAD - _N
    x1 = jnp.pad(boxes[:, 0], (0, pad)).reshape(_ROWS, _COLS)
    y1 = jnp.pad(boxes[:, 1], (0, pad)).reshape(_ROWS, _COLS)
    x2 = jnp.pad(boxes[:, 2], (0, pad)).reshape(_ROWS, _COLS)
    y2 = jnp.pad(boxes[:, 3], (0, pad)).reshape(_ROWS, _COLS)
    sc = jnp.pad(scores, (0, pad), constant_values=_NEG_INF).reshape(_ROWS, _COLS)

    ks, xs1, ys1, xs2, ys2 = _block_sort(sc, x1, y1, x2, y2)

    b0 = jnp.broadcast_to(
        jnp.concatenate([boxes[0], scores[0:1]])[:, None], (5, _COLS))
    b0 = jnp.pad(b0, ((0, _BR - 5), (0, 0)))

    vreg = jax.ShapeDtypeStruct((_BR, _COLS), jnp.float32)
    f32s = pltpu.VMEM((_BR, _COLS), jnp.float32)
    outs = pl.pallas_call(
        _scan_body,
        out_shape=[vreg] * 5,
        scratch_shapes=[
            pltpu.VMEM((1, _COLS), jnp.float32),
            pltpu.VMEM((1, _COLS), jnp.int32),
            pltpu.VMEM((1, _COLS), jnp.float32),
            pltpu.VMEM((1, _COLS), jnp.float32),
            pltpu.VMEM((1, _COLS), jnp.float32),
            pltpu.VMEM((1, _COLS), jnp.float32),
            f32s, f32s, f32s, f32s,
            f32s, f32s,
        ],
    )(ks, xs1, ys1, xs2, ys2, b0)
    return jnp.stack([o.reshape(_K_SELECT) for o in outs], axis=1)


# speedup vs baseline: 1.1162x; 1.1162x over previous
import jax
import jax.numpy as jnp
from jax.experimental import pallas as pl

_IOU_THRESHOLD = 0.8
_K_SELECT = 1024
_N = 20000
_NB = 20
_BR = 8
_COLS = 128
_BLK = _BR * _COLS
_ROWS = _NB * _BR
_N_PAD = _ROWS * _COLS
_NEG_INF = float("-inf")


def _combine(a, b):
    cond = (a[0] > b[0]) | ((a[0] == b[0]) & (a[1] < b[1]))
    return tuple(jnp.where(cond, x, y) for x, y in zip(a, b))


def _nms_body(x1_ref, y1_ref, x2_ref, y2_ref, sc_ref, out_ref):
    lane1 = jax.lax.broadcasted_iota(jnp.int32, (1, _COLS), 1)
    f = (jax.lax.broadcasted_iota(jnp.int32, (_BR, _COLS), 0) * _COLS
         + jax.lax.broadcasted_iota(jnp.int32, (_BR, _COLS), 1))

    def blk(ref, b):
        return ref[pl.ds(_BR * b, _BR), :]

    def body(i, masked):
        ts = [(masked[b], f + _BLK * b, blk(x1_ref, b), blk(y1_ref, b),
               blk(x2_ref, b), blk(y2_ref, b), blk(sc_ref, b))
              for b in range(_NB)]
        while len(ts) > 1:
            nxt = [_combine(ts[j], ts[j + 1]) for j in range(0, len(ts) - 1, 2)]
            if len(ts) % 2:
                nxt.append(ts[-1])
            ts = nxt
        t = ts[0]
        for ax, sh in ((0, 4), (0, 2), (0, 1), (1, 64), (1, 32), (1, 16),
                       (1, 8), (1, 4), (1, 2), (1, 1)):
            t = _combine(t, tuple(jnp.roll(x, sh, axis=ax) for x in t))
        mV, mP, mX1, mY1, mX2, mY2, mS = t

        row = (jnp.where(lane1 == 0, mX1[0:1, :], 0.0)
               + jnp.where(lane1 == 1, mY1[0:1, :], 0.0)
               + jnp.where(lane1 == 2, mX2[0:1, :], 0.0)
               + jnp.where(lane1 == 3, mY2[0:1, :], 0.0)
               + jnp.where(lane1 == 4, mS[0:1, :], 0.0))
        out_ref[pl.ds(i, 1), :] = row

        area_i = (mX2 - mX1) * (mY2 - mY1)
        out_masked = []
        for b in range(_NB):
            x1b = blk(x1_ref, b)
            y1b = blk(y1_ref, b)
            x2b = blk(x2_ref, b)
            y2b = blk(y2_ref, b)
            areas_b = (x2b - x1b) * (y2b - y1b)
            xx1 = jnp.maximum(mX1, x1b)
            yy1 = jnp.maximum(mY1, y1b)
            xx2 = jnp.minimum(mX2, x2b)
            yy2 = jnp.minimum(mY2, y2b)
            inter = jnp.maximum(xx2 - xx1, 0.0) * jnp.maximum(yy2 - yy1, 0.0)
            iou = inter / (area_i + areas_b - inter + 1e-8)
            kill = (iou > _IOU_THRESHOLD) | ((f + _BLK * b) == mP)
            out_masked.append(jnp.where(kill, _NEG_INF, masked[b]))
        return tuple(out_masked)

    masked0 = tuple(blk(sc_ref, b) for b in range(_NB))
    jax.lax.fori_loop(0, _K_SELECT, body, masked0)


def kernel(boxes, scores):
    pad = _N_PAD - _N
    x1 = jnp.pad(boxes[:, 0], (0, pad)).reshape(_ROWS, _COLS)
    y1 = jnp.pad(boxes[:, 1], (0, pad)).reshape(_ROWS, _COLS)
    x2 = jnp.pad(boxes[:, 2], (0, pad)).reshape(_ROWS, _COLS)
    y2 = jnp.pad(boxes[:, 3], (0, pad)).reshape(_ROWS, _COLS)
    sc = jnp.pad(scores, (0, pad), constant_values=_NEG_INF).reshape(_ROWS, _COLS)

    out = pl.pallas_call(
        _nms_body,
        out_shape=jax.ShapeDtypeStruct((_K_SELECT, _COLS), jnp.float32),
    )(x1, y1, x2, y2, sc)
    return out[:, :5]


# speedup vs baseline: 1.1417x; 1.0229x over previous
import jax
import jax.numpy as jnp
from jax.experimental import pallas as pl
from jax.experimental.pallas import tpu as pltpu

_IOU_THRESHOLD = 0.8
_K_SELECT = 1024
_N = 20000
_NB = 20
_BR = 8
_COLS = 128
_BLK = _BR * _COLS
_ROWS = _NB * _BR
_N_PAD = _ROWS * _COLS
_NEG_INF = float("-inf")


def _combine(a, b):
    cond = (a[0] > b[0]) | ((a[0] == b[0]) & (a[1] < b[1]))
    return tuple(jnp.where(cond, x, y) for x, y in zip(a, b))


def _nms_body(x1_ref, y1_ref, x2_ref, y2_ref, sc_ref, out_ref, ms_ref):
    lane1 = jax.lax.broadcasted_iota(jnp.int32, (1, _COLS), 1)
    f = (jax.lax.broadcasted_iota(jnp.int32, (_BR, _COLS), 0) * _COLS
         + jax.lax.broadcasted_iota(jnp.int32, (_BR, _COLS), 1))

    def blk(ref, b):
        return ref[pl.ds(_BR * b, _BR), :]

    ms_ref[...] = sc_ref[...]

    def body(i, prev):
        pP, pX1, pY1, pX2, pY2 = prev
        parea = (pX2 - pX1) * (pY2 - pY1)
        acc = None
        for b in range(_NB):
            x1b = blk(x1_ref, b)
            y1b = blk(y1_ref, b)
            x2b = blk(x2_ref, b)
            y2b = blk(y2_ref, b)
            areas_b = (x2b - x1b) * (y2b - y1b)
            xx1 = jnp.maximum(pX1, x1b)
            yy1 = jnp.maximum(pY1, y1b)
            xx2 = jnp.minimum(pX2, x2b)
            yy2 = jnp.minimum(pY2, y2b)
            inter = jnp.maximum(xx2 - xx1, 0.0) * jnp.maximum(yy2 - yy1, 0.0)
            iou = inter / (parea + areas_b - inter + 1e-8)
            kill = (iou > _IOU_THRESHOLD) | ((f + _BLK * b) == pP)
            mb = jnp.where(kill, _NEG_INF, ms_ref[pl.ds(_BR * b, _BR), :])
            ms_ref[pl.ds(_BR * b, _BR), :] = mb
            tb = (mb, f + _BLK * b, x1b, y1b, x2b, y2b, blk(sc_ref, b))
            acc = tb if acc is None else _combine(acc, tb)
        for ax, sh in ((0, 4), (0, 2), (0, 1), (1, 64), (1, 32), (1, 16),
                       (1, 8), (1, 4), (1, 2), (1, 1)):
            acc = _combine(acc, tuple(jnp.roll(x, sh, axis=ax) for x in acc))
        _, mP, mX1, mY1, mX2, mY2, mS = acc

        row = (jnp.where(lane1 == 0, mX1[0:1, :], 0.0)
               + jnp.where(lane1 == 1, mY1[0:1, :], 0.0)
               + jnp.where(lane1 == 2, mX2[0:1, :], 0.0)
               + jnp.where(lane1 == 3, mY2[0:1, :], 0.0)
               + jnp.where(lane1 == 4, mS[0:1, :], 0.0))
        out_ref[pl.ds(i, 1), :] = row
        return (mP, mX1, mY1, mX2, mY2)

    far = jnp.full((_BR, _COLS), -1.0e30, jnp.float32)
    prev0 = (jnp.full((_BR, _COLS), -1, jnp.int32), far, far, far, far)
    jax.lax.fori_loop(0, _K_SELECT, body, prev0)


def kernel(boxes, scores):
    pad = _N_PAD - _N
    x1 = jnp.pad(boxes[:, 0], (0, pad)).reshape(_ROWS, _COLS)
    y1 = jnp.pad(boxes[:, 1], (0, pad)).reshape(_ROWS, _COLS)
    x2 = jnp.pad(boxes[:, 2], (0, pad)).reshape(_ROWS, _COLS)
    y2 = jnp.pad(boxes[:, 3], (0, pad)).reshape(_ROWS, _COLS)
    sc = jnp.pad(scores, (0, pad), constant_values=_NEG_INF).reshape(_ROWS, _COLS)

    out = pl.pallas_call(
        _nms_body,
        out_shape=jax.ShapeDtypeStruct((_K_SELECT, _COLS), jnp.float32),
        scratch_shapes=[pltpu.VMEM((_ROWS, _COLS), jnp.float32)],
    )(x1, y1, x2, y2, sc)
    return out[:, :5]
